# baseline (device time: 292504 ns/iter reference)
import jax
import jax.numpy as jnp
from jax import lax
from jax.experimental import pallas as pl
from jax.experimental.pallas import tpu as pltpu

N = 32
H = N // 2
B = 2
SQL = 128
D = 512
HL = 8
DH = 64
SKV = 128

_PLANE = {(0, 0): 0, (1, 0): 1, (1, 1): 2, (0, 1): 3,
          (0, 2): 4, (1, 2): 5, (1, 3): 6, (0, 3): 7}
_PATH = [(0, 0), (1, 0), (2, 0), (3, 0), (3, 1), (2, 1), (1, 1), (0, 1),
         (0, 2), (1, 2), (2, 2), (3, 2), (3, 3), (2, 3), (1, 3), (0, 3)]
_CYCLE = [(0, y, z) for (y, z) in _PATH] + [(1, y, z) for (y, z) in reversed(_PATH)]
SIGMA = [z * 8 + _PLANE[(x, y)] for (x, y, z) in _CYCLE]
INV = [0] * N
for _r, _p in enumerate(SIGMA):
    INV[_p] = _r


def kernel(x, Wq, Wo, K_ext, V_ext):
    my = lax.axis_index("i")

    k_loc = lax.dynamic_slice_in_dim(K_ext, my * HL, HL, axis=2)
    v_loc = lax.dynamic_slice_in_dim(V_ext, my * HL, HL, axis=2)
    k_loc = jnp.transpose(k_loc, (0, 2, 1, 3)).astype(jnp.bfloat16)
    v_loc = jnp.transpose(v_loc, (0, 2, 1, 3)).astype(jnp.bfloat16)
    wq16 = Wq.astype(jnp.bfloat16)
    wo16 = Wo.astype(jnp.bfloat16)
    x16 = x.astype(jnp.bfloat16)

    sigma = jnp.array(SIGMA, jnp.int32)
    inv = jnp.array(INV, jnp.int32)
    rank = inv[my]
    scal = jnp.stack([rank,
                      sigma[lax.rem(rank + 1, N)],
                      sigma[lax.rem(rank + N - 1, N)]]).astype(jnp.int32)

    def body(scal_ref, x_ref, wq_ref, wo_ref, k_ref, v_ref, out_ref,
             xfull, part, attn, rsbuf, sbR, sbL,
             ag_send, ag_recv, rs_send, rs_recv):
        right = scal_ref[1]
        left = scal_ref[2]

        def ag_copy(src_slot, dst_slot, sem_i, dev):
            return pltpu.make_async_remote_copy(
                src_ref=xfull.at[:, pl.ds(src_slot * SQL, SQL), :],
                dst_ref=xfull.at[:, pl.ds(dst_slot * SQL, SQL), :],
                send_sem=ag_send.at[sem_i], recv_sem=ag_recv.at[sem_i],
                device_id=(dev,), device_id_type=pl.DeviceIdType.MESH)

        def rs_copy(src, dst_i, sem_i, dev):
            return pltpu.make_async_remote_copy(
                src_ref=src, dst_ref=rsbuf.at[dst_i],
                send_sem=rs_send.at[sem_i], recv_sem=rs_recv.at[sem_i],
                device_id=(dev,), device_id_type=pl.DeviceIdType.MESH)

        def pslot(s):
            return part[:, pl.ds(s * SQL, SQL), :].astype(jnp.float32)

        def compute_rows(s0, nch):
            rows = nch * SQL

            def bstep(b, carry):
                xb = xfull[b, pl.ds(s0 * SQL, rows), :]
                qf = jnp.dot(xb, wq_ref[...],
                             preferred_element_type=jnp.float32
                             ).astype(jnp.bfloat16)
                for hh in range(HL):
                    q = qf[:, hh * DH:(hh + 1) * DH]
                    s = lax.dot_general(
                        q, k_ref[b, hh], (((1,), (1,)), ((), ())),
                        preferred_element_type=jnp.float32) * 0.125
                    m = jnp.max(s, axis=-1, keepdims=True)
                    p = jnp.exp(s - m)
                    l = jnp.sum(p, axis=-1, keepdims=True)
                    pv = (p / l).astype(jnp.bfloat16)
                    a = jnp.dot(pv, v_ref[b, hh],
                                preferred_element_type=jnp.float32)
                    attn[0:rows, hh * DH:(hh + 1) * DH] = a.astype(jnp.bfloat16)
                part[b, pl.ds(s0 * SQL, rows), :] = jnp.dot(
                    attn[0:rows, :], wo_ref[...],
                    preferred_element_type=jnp.float32).astype(jnp.bfloat16)
                return carry

            lax.fori_loop(0, B, bstep, 0)

        barrier = pltpu.get_barrier_semaphore()
        pl.semaphore_signal(barrier, inc=1, device_id=(left,),
                            device_id_type=pl.DeviceIdType.MESH)
        pl.semaphore_signal(barrier, inc=1, device_id=(right,),
                            device_id_type=pl.DeviceIdType.MESH)
        pl.semaphore_wait(barrier, 2)

        xfull[:, (N - 1) * SQL:N * SQL, :] = x_ref[...]

        def ag_step(t, carry):
            sR = ag_copy(31 - t, 30 - t, t, right)
            sL = ag_copy(lax.rem(t - 1 + N, N), t, H + t, left)
            sR.start()
            sL.start()
            rR = ag_copy(30 - t, 30 - t, t, left)
            rL = ag_copy(t, t, H + t, right)
            sR.wait_send()
            sL.wait_send()
            rR.wait_recv()
            rL.wait_recv()
            return carry

        lax.fori_loop(0, H - 1, ag_step, 0)
        tf = H - 1
        sR = ag_copy(31 - tf, 30 - tf, tf, right)
        sR.start()
        rR = ag_copy(30 - tf, 30 - tf, tf, left)
        sR.wait_send()
        rR.wait_recv()

        compute_rows(13, 6)

        sbR[...] = pslot(H - 1).astype(jnp.bfloat16)
        sbL[...] = pslot(H).astype(jnp.bfloat16)
        r0 = rs_copy(sbR, 0, 0, right)
        l0 = rs_copy(sbL, H, H, left)
        r0.start()
        l0.start()
        r0.wait()
        l0.wait()

        def rs_step(t, carry):
            sbR[...] = (rsbuf[t - 1].astype(jnp.float32)
                        + pslot(H - 1 - t)).astype(jnp.bfloat16)
            sbL[...] = (rsbuf[H + t - 1].astype(jnp.float32)
                        + pslot(H + t)).astype(jnp.bfloat16)
            rr = rs_copy(sbR, t, t, right)
            ll = rs_copy(sbL, H + t, H + t, left)
            rr.start()
            ll.start()
            compute_rows(jnp.maximum(13 - t, 0), 1)
            compute_rows(jnp.minimum(18 + t, N - 1), 1)
            rr.wait()
            ll.wait()
            return carry

        lax.fori_loop(1, H - 1, rs_step, 0)
        sbR[...] = (rsbuf[H - 2].astype(jnp.float32) + pslot(0)
                    ).astype(jnp.bfloat16)
        rf = rs_copy(sbR, H - 1, H - 1, right)
        rf.start()
        rf.wait()

        out_ref[...] = (rsbuf[H - 1].astype(jnp.float32)
                        + rsbuf[2 * H - 2].astype(jnp.float32)
                        + pslot(N - 1))

    return pl.pallas_call(
        body,
        out_shape=jax.ShapeDtypeStruct((B, SQL, D), jnp.float32),
        in_specs=[pl.BlockSpec(memory_space=pltpu.SMEM)]
        + [pl.BlockSpec(memory_space=pltpu.VMEM)] * 5,
        out_specs=pl.BlockSpec(memory_space=pltpu.VMEM),
        scratch_shapes=[
            pltpu.VMEM((B, N * SQL, D), jnp.bfloat16),
            pltpu.VMEM((B, N * SQL, D), jnp.bfloat16),
            pltpu.VMEM((N * SQL, D), jnp.bfloat16),
            pltpu.VMEM((N - 1, B, SQL, D), jnp.bfloat16),
            pltpu.VMEM((B, SQL, D), jnp.bfloat16),
            pltpu.VMEM((B, SQL, D), jnp.bfloat16),
            pltpu.SemaphoreType.DMA((N - 1,)),
            pltpu.SemaphoreType.DMA((N - 1,)),
            pltpu.SemaphoreType.DMA((N - 1,)),
            pltpu.SemaphoreType.DMA((N - 1,)),
        ],
        compiler_params=pltpu.CompilerParams(
            collective_id=0, vmem_limit_bytes=60 * 1024 * 1024),
    )(scal, x16, wq16, wo16, k_loc, v_loc)


# device time: 207854 ns/iter; 1.4073x vs baseline; 1.4073x over previous
import jax
import jax.numpy as jnp
from jax import lax
from jax.experimental import pallas as pl
from jax.experimental.pallas import tpu as pltpu

N = 32
H = N // 2
B = 2
SQL = 128
D = 512
HL = 8
DH = 64
SKV = 128

_PLANE = {(0, 0): 0, (1, 0): 1, (1, 1): 2, (0, 1): 3,
          (0, 2): 4, (1, 2): 5, (1, 3): 6, (0, 3): 7}
_PATH = [(0, 0), (1, 0), (2, 0), (3, 0), (3, 1), (2, 1), (1, 1), (0, 1),
         (0, 2), (1, 2), (2, 2), (3, 2), (3, 3), (2, 3), (1, 3), (0, 3)]
_CYCLE = [(0, y, z) for (y, z) in _PATH] + [(1, y, z) for (y, z) in reversed(_PATH)]
SIGMA = [z * 8 + _PLANE[(x, y)] for (x, y, z) in _CYCLE]
INV = [0] * N
for _r, _p in enumerate(SIGMA):
    INV[_p] = _r


def kernel(x, Wq, Wo, K_ext, V_ext):
    my = lax.axis_index("i")

    k_loc = lax.dynamic_slice_in_dim(K_ext, my * HL, HL, axis=2)
    v_loc = lax.dynamic_slice_in_dim(V_ext, my * HL, HL, axis=2)
    k_loc = jnp.transpose(k_loc, (0, 2, 1, 3)).astype(jnp.bfloat16)
    v_loc = jnp.transpose(v_loc, (0, 2, 1, 3)).astype(jnp.bfloat16)
    kT = jnp.transpose(k_loc, (0, 1, 3, 2))
    zk = jnp.zeros_like(kT[:, 0])
    zv = jnp.zeros_like(v_loc[:, 0])
    k2bd = jnp.stack(
        [jnp.concatenate(
            [jnp.concatenate([kT[:, 2 * g], zk], axis=-1),
             jnp.concatenate([zk, kT[:, 2 * g + 1]], axis=-1)], axis=1)
         for g in range(HL // 2)], axis=1)
    v2bd = jnp.stack(
        [jnp.concatenate(
            [jnp.concatenate([v_loc[:, 2 * g], zv], axis=-1),
             jnp.concatenate([zv, v_loc[:, 2 * g + 1]], axis=-1)], axis=1)
         for g in range(HL // 2)], axis=1)
    wq16 = Wq.astype(jnp.bfloat16)
    wo16 = Wo.astype(jnp.bfloat16)
    x16 = x.astype(jnp.bfloat16)

    sigma = jnp.array(SIGMA, jnp.int32)
    inv = jnp.array(INV, jnp.int32)
    rank = inv[my]
    scal = jnp.stack([rank,
                      sigma[lax.rem(rank + 1, N)],
                      sigma[lax.rem(rank + N - 1, N)]]).astype(jnp.int32)

    def body(scal_ref, x_ref, wq_ref, wo_ref, k_ref, v_ref, out_ref,
             xfull, part, attn, rsbuf, sbR, sbL,
             ag_send, ag_recv, rs_send, rs_recv):
        right = scal_ref[1]
        left = scal_ref[2]

        def ag_copy(src_slot, dst_slot, sem_i, dev):
            return pltpu.make_async_remote_copy(
                src_ref=xfull.at[:, pl.ds(src_slot * SQL, SQL), :],
                dst_ref=xfull.at[:, pl.ds(dst_slot * SQL, SQL), :],
                send_sem=ag_send.at[sem_i], recv_sem=ag_recv.at[sem_i],
                device_id=(dev,), device_id_type=pl.DeviceIdType.MESH)

        def rs_copy(src, dst_i, sem_i, dev):
            return pltpu.make_async_remote_copy(
                src_ref=src, dst_ref=rsbuf.at[dst_i],
                send_sem=rs_send.at[sem_i], recv_sem=rs_recv.at[sem_i],
                device_id=(dev,), device_id_type=pl.DeviceIdType.MESH)

        def pslot(s):
            return part[:, pl.ds(s * SQL, SQL), :].astype(jnp.float32)

        def compute_rows(s0, nch):
            rows = nch * SQL

            def bstep(b, carry):
                xb = xfull[b, pl.ds(s0 * SQL, rows), :]
                qf = jnp.dot(xb, wq_ref[...],
                             preferred_element_type=jnp.float32
                             ).astype(jnp.bfloat16)
                for g in range(HL // 2):
                    q2 = qf[:, 2 * g * DH:2 * (g + 1) * DH]
                    s = jnp.dot(q2, k_ref[b, g],
                                preferred_element_type=jnp.float32) * 0.125
                    m = jnp.max(s, axis=-1, keepdims=True)
                    p = jnp.exp(s - m)
                    la = jnp.sum(p[:, :SKV], axis=-1, keepdims=True)
                    lb = jnp.sum(p[:, SKV:], axis=-1, keepdims=True)
                    pv = jnp.concatenate(
                        [p[:, :SKV] / la, p[:, SKV:] / lb],
                        axis=-1).astype(jnp.bfloat16)
                    a2 = jnp.dot(pv, v_ref[b, g],
                                 preferred_element_type=jnp.float32)
                    attn[0:rows, 2 * g * DH:2 * (g + 1) * DH] = (
                        a2.astype(jnp.bfloat16))
                part[b, pl.ds(s0 * SQL, rows), :] = jnp.dot(
                    attn[0:rows, :], wo_ref[...],
                    preferred_element_type=jnp.float32).astype(jnp.bfloat16)
                return carry

            lax.fori_loop(0, B, bstep, 0)

        barrier = pltpu.get_barrier_semaphore()
        pl.semaphore_signal(barrier, inc=1, device_id=(left,),
                            device_id_type=pl.DeviceIdType.MESH)
        pl.semaphore_signal(barrier, inc=1, device_id=(right,),
                            device_id_type=pl.DeviceIdType.MESH)
        pl.semaphore_wait(barrier, 2)

        xfull[:, (N - 1) * SQL:N * SQL, :] = x_ref[...]

        def ag_step(t, carry):
            sR = ag_copy(31 - t, 30 - t, t, right)
            sL = ag_copy(lax.rem(t - 1 + N, N), t, H + t, left)
            sR.start()
            sL.start()
            rR = ag_copy(30 - t, 30 - t, t, left)
            rL = ag_copy(t, t, H + t, right)
            sR.wait_send()
            sL.wait_send()
            rR.wait_recv()
            rL.wait_recv()
            return carry

        lax.fori_loop(0, H - 1, ag_step, 0)
        tf = H - 1
        sR = ag_copy(31 - tf, 30 - tf, tf, right)
        sR.start()
        rR = ag_copy(30 - tf, 30 - tf, tf, left)
        sR.wait_send()
        rR.wait_recv()

        compute_rows(0, N)

        sbR[...] = pslot(H - 1).astype(jnp.bfloat16)
        sbL[...] = pslot(H).astype(jnp.bfloat16)
        r0 = rs_copy(sbR, 0, 0, right)
        l0 = rs_copy(sbL, H, H, left)
        r0.start()
        l0.start()
        r0.wait()
        l0.wait()

        def rs_step(t, carry):
            sbR[...] = (rsbuf[t - 1].astype(jnp.float32)
                        + pslot(H - 1 - t)).astype(jnp.bfloat16)
            sbL[...] = (rsbuf[H + t - 1].astype(jnp.float32)
                        + pslot(H + t)).astype(jnp.bfloat16)
            rr = rs_copy(sbR, t, t, right)
            ll = rs_copy(sbL, H + t, H + t, left)
            rr.start()
            ll.start()
            rr.wait()
            ll.wait()
            return carry

        lax.fori_loop(1, H - 1, rs_step, 0)
        sbR[...] = (rsbuf[H - 2].astype(jnp.float32) + pslot(0)
                    ).astype(jnp.bfloat16)
        rf = rs_copy(sbR, H - 1, H - 1, right)
        rf.start()
        rf.wait()

        out_ref[...] = (rsbuf[H - 1].astype(jnp.float32)
                        + rsbuf[2 * H - 2].astype(jnp.float32)
                        + pslot(N - 1))

    return pl.pallas_call(
        body,
        out_shape=jax.ShapeDtypeStruct((B, SQL, D), jnp.float32),
        in_specs=[pl.BlockSpec(memory_space=pltpu.SMEM)]
        + [pl.BlockSpec(memory_space=pltpu.VMEM)] * 5,
        out_specs=pl.BlockSpec(memory_space=pltpu.VMEM),
        scratch_shapes=[
            pltpu.VMEM((B, N * SQL, D), jnp.bfloat16),
            pltpu.VMEM((B, N * SQL, D), jnp.bfloat16),
            pltpu.VMEM((N * SQL, D), jnp.bfloat16),
            pltpu.VMEM((N - 1, B, SQL, D), jnp.bfloat16),
            pltpu.VMEM((B, SQL, D), jnp.bfloat16),
            pltpu.VMEM((B, SQL, D), jnp.bfloat16),
            pltpu.SemaphoreType.DMA((N - 1,)),
            pltpu.SemaphoreType.DMA((N - 1,)),
            pltpu.SemaphoreType.DMA((N - 1,)),
            pltpu.SemaphoreType.DMA((N - 1,)),
        ],
        compiler_params=pltpu.CompilerParams(
            collective_id=0, vmem_limit_bytes=60 * 1024 * 1024),
    )(scal, x16, wq16, wo16, k2bd, v2bd)


# device time: 188959 ns/iter; 1.5480x vs baseline; 1.1000x over previous
import jax
import jax.numpy as jnp
from jax import lax
from jax.experimental import pallas as pl
from jax.experimental.pallas import tpu as pltpu

N = 32
H = N // 2
B = 2
SQL = 128
D = 512
HL = 8
DH = 64
SKV = 128

_PLANE = {(0, 0): 0, (1, 0): 1, (1, 1): 2, (0, 1): 3,
          (0, 2): 4, (1, 2): 5, (1, 3): 6, (0, 3): 7}
_PATH = [(0, 0), (1, 0), (2, 0), (3, 0), (3, 1), (2, 1), (1, 1), (0, 1),
         (0, 2), (1, 2), (2, 2), (3, 2), (3, 3), (2, 3), (1, 3), (0, 3)]
_CYCLE = [(0, y, z) for (y, z) in _PATH] + [(1, y, z) for (y, z) in reversed(_PATH)]
SIGMA = [z * 8 + _PLANE[(x, y)] for (x, y, z) in _CYCLE]
INV = [0] * N
for _r, _p in enumerate(SIGMA):
    INV[_p] = _r


def kernel(x, Wq, Wo, K_ext, V_ext):
    my = lax.axis_index("i")

    k_loc = lax.dynamic_slice_in_dim(K_ext, my * HL, HL, axis=2)
    v_loc = lax.dynamic_slice_in_dim(V_ext, my * HL, HL, axis=2)
    k_loc = jnp.transpose(k_loc, (0, 2, 1, 3)).astype(jnp.bfloat16)
    v_loc = jnp.transpose(v_loc, (0, 2, 1, 3)).astype(jnp.bfloat16)
    kT = jnp.transpose(k_loc, (0, 1, 3, 2))
    zk = jnp.zeros_like(kT[:, 0])
    zv = jnp.zeros_like(v_loc[:, 0])
    k2bd = jnp.stack(
        [jnp.concatenate(
            [jnp.concatenate([kT[:, 2 * g], zk], axis=-1),
             jnp.concatenate([zk, kT[:, 2 * g + 1]], axis=-1)], axis=1)
         for g in range(HL // 2)], axis=1)
    v2bd = jnp.stack(
        [jnp.concatenate(
            [jnp.concatenate([v_loc[:, 2 * g], zv], axis=-1),
             jnp.concatenate([zv, v_loc[:, 2 * g + 1]], axis=-1)], axis=1)
         for g in range(HL // 2)], axis=1)
    wq16 = Wq.astype(jnp.bfloat16)
    wo16 = Wo.astype(jnp.bfloat16)
    x16 = x.astype(jnp.bfloat16)

    sigma = jnp.array(SIGMA, jnp.int32)
    inv = jnp.array(INV, jnp.int32)
    rank = inv[my]
    scal = jnp.stack([rank,
                      sigma[lax.rem(rank + 1, N)],
                      sigma[lax.rem(rank + N - 1, N)]]).astype(jnp.int32)

    def body(scal_ref, x_ref, wq_ref, wo_ref, k_ref, v_ref, out_ref,
             xfull, part, attn, rsbuf, sbR, sbL,
             ag_send, ag_recv, rs_send, rs_recv):
        right = scal_ref[1]
        left = scal_ref[2]

        def ag_copy(src_slot, dst_slot, sem_i, dev):
            return pltpu.make_async_remote_copy(
                src_ref=xfull.at[:, pl.ds(src_slot * SQL, SQL), :],
                dst_ref=xfull.at[:, pl.ds(dst_slot * SQL, SQL), :],
                send_sem=ag_send.at[sem_i], recv_sem=ag_recv.at[sem_i],
                device_id=(dev,), device_id_type=pl.DeviceIdType.MESH)

        def rs_copy(src, dst_i, sem_i, dev):
            return pltpu.make_async_remote_copy(
                src_ref=src, dst_ref=rsbuf.at[dst_i],
                send_sem=rs_send.at[sem_i], recv_sem=rs_recv.at[sem_i],
                device_id=(dev,), device_id_type=pl.DeviceIdType.MESH)

        def pslot(s):
            return part[:, pl.ds(s * SQL, SQL), :].astype(jnp.float32)

        def compute_rows(s0, nch):
            rows = nch * SQL

            def bstep(b, carry):
                xb = xfull[b, pl.ds(s0 * SQL, rows), :]
                qf = jnp.dot(xb, wq_ref[...],
                             preferred_element_type=jnp.float32
                             ).astype(jnp.bfloat16)
                for g in range(HL // 2):
                    q2 = qf[:, 2 * g * DH:2 * (g + 1) * DH]
                    s = jnp.dot(q2, k_ref[b, g],
                                preferred_element_type=jnp.float32) * 0.125
                    m = jnp.max(s, axis=-1, keepdims=True)
                    p = jnp.exp(s - m)
                    la = jnp.sum(p[:, :SKV], axis=-1, keepdims=True)
                    lb = jnp.sum(p[:, SKV:], axis=-1, keepdims=True)
                    pv = jnp.concatenate(
                        [p[:, :SKV] / la, p[:, SKV:] / lb],
                        axis=-1).astype(jnp.bfloat16)
                    a2 = jnp.dot(pv, v_ref[b, g],
                                 preferred_element_type=jnp.float32)
                    attn[0:rows, 2 * g * DH:2 * (g + 1) * DH] = (
                        a2.astype(jnp.bfloat16))
                part[b, pl.ds(s0 * SQL, rows), :] = jnp.dot(
                    attn[0:rows, :], wo_ref[...],
                    preferred_element_type=jnp.float32).astype(jnp.bfloat16)
                return carry

            lax.fori_loop(0, B, bstep, 0)

        barrier = pltpu.get_barrier_semaphore()
        pl.semaphore_signal(barrier, inc=1, device_id=(left,),
                            device_id_type=pl.DeviceIdType.MESH)
        pl.semaphore_signal(barrier, inc=1, device_id=(right,),
                            device_id_type=pl.DeviceIdType.MESH)
        pl.semaphore_wait(barrier, 2)

        xfull[:, (N - 1) * SQL:N * SQL, :] = x_ref[...]

        def ag_step(t, carry):
            sR = ag_copy(31 - t, 30 - t, t, right)
            sL = ag_copy(lax.rem(t - 1 + N, N), t, H + t, left)
            sR.start()
            sL.start()
            rR = ag_copy(30 - t, 30 - t, t, left)
            rL = ag_copy(t, t, H + t, right)
            sR.wait_send()
            sL.wait_send()
            rR.wait_recv()
            rL.wait_recv()
            return carry

        lax.fori_loop(0, H - 1, ag_step, 0)
        tf = H - 1
        sR = ag_copy(31 - tf, 30 - tf, tf, right)
        sR.start()
        rR = ag_copy(30 - tf, 30 - tf, tf, left)
        sR.wait_send()
        rR.wait_recv()

        compute_rows(13, 6)

        sbR[...] = pslot(H - 1).astype(jnp.bfloat16)
        sbL[...] = pslot(H).astype(jnp.bfloat16)
        r0 = rs_copy(sbR, 0, 0, right)
        l0 = rs_copy(sbL, H, H, left)
        r0.start()
        l0.start()
        r0.wait()
        l0.wait()

        def rs_step(t, carry):
            sbR[...] = (rsbuf[t - 1].astype(jnp.float32)
                        + pslot(H - 1 - t)).astype(jnp.bfloat16)
            sbL[...] = (rsbuf[H + t - 1].astype(jnp.float32)
                        + pslot(H + t)).astype(jnp.bfloat16)
            rr = rs_copy(sbR, t, t, right)
            ll = rs_copy(sbL, H + t, H + t, left)
            rr.start()
            ll.start()
            cstart = jnp.where(lax.rem(t, 2) == 1,
                               jnp.maximum(12 - t, 0),
                               jnp.minimum(17 + t, N - 2))
            compute_rows(cstart, 2)
            rr.wait()
            ll.wait()
            return carry

        lax.fori_loop(1, H - 1, rs_step, 0)
        sbR[...] = (rsbuf[H - 2].astype(jnp.float32) + pslot(0)
                    ).astype(jnp.bfloat16)
        rf = rs_copy(sbR, H - 1, H - 1, right)
        rf.start()
        rf.wait()

        out_ref[...] = (rsbuf[H - 1].astype(jnp.float32)
                        + rsbuf[2 * H - 2].astype(jnp.float32)
                        + pslot(N - 1))

    return pl.pallas_call(
        body,
        out_shape=jax.ShapeDtypeStruct((B, SQL, D), jnp.float32),
        in_specs=[pl.BlockSpec(memory_space=pltpu.SMEM)]
        + [pl.BlockSpec(memory_space=pltpu.VMEM)] * 5,
        out_specs=pl.BlockSpec(memory_space=pltpu.VMEM),
        scratch_shapes=[
            pltpu.VMEM((B, N * SQL, D), jnp.bfloat16),
            pltpu.VMEM((B, N * SQL, D), jnp.bfloat16),
            pltpu.VMEM((N * SQL, D), jnp.bfloat16),
            pltpu.VMEM((N - 1, B, SQL, D), jnp.bfloat16),
            pltpu.VMEM((B, SQL, D), jnp.bfloat16),
            pltpu.VMEM((B, SQL, D), jnp.bfloat16),
            pltpu.SemaphoreType.DMA((N - 1,)),
            pltpu.SemaphoreType.DMA((N - 1,)),
            pltpu.SemaphoreType.DMA((N - 1,)),
            pltpu.SemaphoreType.DMA((N - 1,)),
        ],
        compiler_params=pltpu.CompilerParams(
            collective_id=0, vmem_limit_bytes=60 * 1024 * 1024),
    )(scal, x16, wq16, wo16, k2bd, v2bd)
